# Initial kernel scaffold; baseline (speedup 1.0000x reference)
import jax
import jax.numpy as jnp
from jax.experimental import pallas as pl


def kernel(input_values, attention_mask, task_embed, exp_down_w, exp_down_b, exp_up_w, exp_up_b, r_w1, r_b1, r_w2, r_b2, head_asr_w, head_asr_b, head_ser_w, head_ser_b, head_sr_w, head_sr_b):
    raise NotImplementedError("write your pallas kernel here")


# speedup vs baseline: 2.4385x; 2.4385x over previous
import jax
import jax.numpy as jnp
from jax.experimental import pallas as pl
from jax.experimental.pallas import tpu as pltpu

B, T, H = 8, 512, 1024
E, BN, RH = 8, 256, 256
VOCAB, EMO, SPK = 1000, 8, 1000
TOP_K = 2
TEMP = 2.0
VPAD = 1024


def _gelu(x):
    return 0.5 * x * (1.0 + jax.lax.erf(x * 0.7071067811865476))


def _router_kernel(h_ref, m_ref, te_ref, w1_ref, b1_ref, w2_ref, b2_ref,
                   pooled_ref, topi_ref, gates_ref):
    m = m_ref[...]
    den = jnp.maximum(m.sum(axis=1, keepdims=True), 1.0)
    num = (h_ref[...] * m[:, :, None]).sum(axis=1)
    pooled = num / den
    pooled_ref[...] = pooled
    for rid in range(3):
        x = pooled + te_ref[rid][None, :]
        rh = jnp.dot(x.astype(jnp.bfloat16), w1_ref[rid],
                     preferred_element_type=jnp.float32) + b1_ref[rid][None, :]
        rh = _gelu(rh)
        logits = (jnp.dot(rh.astype(jnp.bfloat16), w2_ref[rid],
                          preferred_element_type=jnp.float32)
                  + b2_ref[rid][None, :]) / TEMP
        i1 = jnp.argmax(logits, axis=-1).astype(jnp.int32)
        v1 = jnp.max(logits, axis=-1)
        col = jax.lax.broadcasted_iota(jnp.int32, (B, E), 1)
        masked = jnp.where(col == i1[:, None], -jnp.inf, logits)
        i2 = jnp.argmax(masked, axis=-1).astype(jnp.int32)
        v2 = jnp.max(masked, axis=-1)
        g1 = jax.nn.sigmoid(v1 - v2)
        topi_ref[rid, :, 0] = i1
        topi_ref[rid, :, 1] = i2
        gates_ref[rid, :, 0] = g1
        gates_ref[rid, :, 1] = 1.0 - g1


def _expert_kernel(topi_ref, gates_ref,
                   h_ref, m_ref, te_ref, wd_ref, db_ref, wu_ref, ub_ref,
                   asr_ref, pmix_ref):
    b = pl.program_id(0)
    rid = pl.program_id(1)
    k = pl.program_id(2)
    g = gates_ref[(rid * B + b) * 2 + k]
    x = h_ref[0] + te_ref[0, 0][None, :]
    d = jnp.dot(x.astype(jnp.bfloat16), wd_ref[0],
                preferred_element_type=jnp.float32) + db_ref[0, 0][None, :]
    a = _gelu(d)
    u = jnp.dot(a.astype(jnp.bfloat16), wu_ref[0],
                preferred_element_type=jnp.float32) + ub_ref[0, 0][None, :]
    c = g * u

    @pl.when(rid == 0)
    def _():
        @pl.when(k == 0)
        def _():
            asr_ref[0] = h_ref[0] + c

        @pl.when(k == 1)
        def _():
            asr_ref[0] += c

    @pl.when(rid > 0)
    def _():
        pc = (c * m_ref[0, 0][:, None]).sum(axis=0)

        @pl.when(k == 0)
        def _():
            pmix_ref[0, 0, 0] = pc

        @pl.when(k == 1)
        def _():
            pmix_ref[0, 0, 0] += pc


def _heads_kernel(asr_ref, wasr_ref, basr_ref, m_ref, pooled_ref, pmix_ref,
                  wser_ref, bser_ref, wsr_ref, bsr_ref,
                  la_ref, lser_ref, lsr_ref):
    b = pl.program_id(0)
    la_ref[0] = jnp.dot(asr_ref[0].astype(jnp.bfloat16), wasr_ref[...],
                        preferred_element_type=jnp.float32) + basr_ref[0][None, :]

    @pl.when(b == 0)
    def _():
        den = jnp.maximum(m_ref[...].sum(axis=1, keepdims=True), 1.0)
        ps = pooled_ref[...] + pmix_ref[0] / den
        pr = pooled_ref[...] + pmix_ref[1] / den
        lser_ref[...] = jnp.dot(ps.astype(jnp.bfloat16), wser_ref[...],
                                preferred_element_type=jnp.float32) + bser_ref[0][None, :]
        lsr_ref[...] = jnp.dot(pr.astype(jnp.bfloat16), wsr_ref[...],
                               preferred_element_type=jnp.float32) + bsr_ref[0][None, :]


def kernel(input_values, attention_mask, task_embed, exp_down_w, exp_down_b,
           exp_up_w, exp_up_b, r_w1, r_b1, r_w2, r_b2, head_asr_w, head_asr_b,
           head_ser_w, head_ser_b, head_sr_w, head_sr_b):
    h = input_values
    mask = attention_mask
    f32 = jnp.float32
    bf16 = jnp.bfloat16

    pooled, topi, gates = pl.pallas_call(
        _router_kernel,
        grid=(1,),
        in_specs=[
            pl.BlockSpec((B, T, H), lambda i: (0, 0, 0)),
            pl.BlockSpec((B, T), lambda i: (0, 0)),
            pl.BlockSpec((3, H), lambda i: (0, 0)),
            pl.BlockSpec((3, H, RH), lambda i: (0, 0, 0)),
            pl.BlockSpec((3, RH), lambda i: (0, 0)),
            pl.BlockSpec((3, RH, E), lambda i: (0, 0, 0)),
            pl.BlockSpec((3, E), lambda i: (0, 0)),
        ],
        out_specs=[
            pl.BlockSpec((B, H), lambda i: (0, 0)),
            pl.BlockSpec((3, B, 2), lambda i: (0, 0, 0)),
            pl.BlockSpec((3, B, 2), lambda i: (0, 0, 0)),
        ],
        out_shape=[
            jax.ShapeDtypeStruct((B, H), f32),
            jax.ShapeDtypeStruct((3, B, 2), jnp.int32),
            jax.ShapeDtypeStruct((3, B, 2), f32),
        ],
    )(h, mask, task_embed, r_w1.astype(bf16), r_b1, r_w2.astype(bf16), r_b2)

    topi_flat = topi.reshape(-1)
    gates_flat = gates.reshape(-1)

    asr_states, pmix = pl.pallas_call(
        _expert_kernel,
        grid_spec=pltpu.PrefetchScalarGridSpec(
            num_scalar_prefetch=2,
            grid=(B, 3, 2),
            in_specs=[
                pl.BlockSpec((1, T, H), lambda b, r, k, ti, gv: (b, 0, 0)),
                pl.BlockSpec((1, 1, T), lambda b, r, k, ti, gv: (b, 0, 0)),
                pl.BlockSpec((1, 1, H), lambda b, r, k, ti, gv: (r, 0, 0)),
                pl.BlockSpec((1, H, BN),
                             lambda b, r, k, ti, gv: (ti[(r * B + b) * 2 + k], 0, 0)),
                pl.BlockSpec((1, 1, BN),
                             lambda b, r, k, ti, gv: (ti[(r * B + b) * 2 + k], 0, 0)),
                pl.BlockSpec((1, BN, H),
                             lambda b, r, k, ti, gv: (ti[(r * B + b) * 2 + k], 0, 0)),
                pl.BlockSpec((1, 1, H),
                             lambda b, r, k, ti, gv: (ti[(r * B + b) * 2 + k], 0, 0)),
            ],
            out_specs=[
                pl.BlockSpec((1, T, H), lambda b, r, k, ti, gv: (b, 0, 0)),
                pl.BlockSpec((1, 1, 1, H),
                             lambda b, r, k, ti, gv: (jnp.maximum(r - 1, 0), b, 0, 0)),
            ],
        ),
        out_shape=[
            jax.ShapeDtypeStruct((B, T, H), f32),
            jax.ShapeDtypeStruct((2, B, 1, H), f32),
        ],
    )(topi_flat, gates_flat, h, mask.reshape(B, 1, T), task_embed.reshape(3, 1, H),
      exp_down_w.astype(bf16), exp_down_b.reshape(E, 1, BN),
      exp_up_w.astype(bf16), exp_up_b.reshape(E, 1, H))
    pmix = pmix.reshape(2, B, H)

    wasr = jnp.zeros((H, VPAD), bf16).at[:, :VOCAB].set(head_asr_w.astype(bf16))
    basr = jnp.zeros((1, VPAD), f32).at[0, :VOCAB].set(head_asr_b)
    wsr = jnp.zeros((H, VPAD), bf16).at[:, :SPK].set(head_sr_w.astype(bf16))
    bsr = jnp.zeros((1, VPAD), f32).at[0, :SPK].set(head_sr_b)

    la, lser, lsr = pl.pallas_call(
        _heads_kernel,
        grid=(B,),
        in_specs=[
            pl.BlockSpec((1, T, H), lambda b: (b, 0, 0)),
            pl.BlockSpec((H, VPAD), lambda b: (0, 0)),
            pl.BlockSpec((1, VPAD), lambda b: (0, 0)),
            pl.BlockSpec((B, T), lambda b: (0, 0)),
            pl.BlockSpec((B, H), lambda b: (0, 0)),
            pl.BlockSpec((2, B, H), lambda b: (0, 0, 0)),
            pl.BlockSpec((H, EMO), lambda b: (0, 0)),
            pl.BlockSpec((1, EMO), lambda b: (0, 0)),
            pl.BlockSpec((H, VPAD), lambda b: (0, 0)),
            pl.BlockSpec((1, VPAD), lambda b: (0, 0)),
        ],
        out_specs=[
            pl.BlockSpec((1, T, VPAD), lambda b: (b, 0, 0)),
            pl.BlockSpec((B, EMO), lambda b: (0, 0)),
            pl.BlockSpec((B, VPAD), lambda b: (0, 0)),
        ],
        out_shape=[
            jax.ShapeDtypeStruct((B, T, VPAD), f32),
            jax.ShapeDtypeStruct((B, EMO), f32),
            jax.ShapeDtypeStruct((B, VPAD), f32),
        ],
    )(asr_states, wasr, basr, mask, pooled, pmix,
      head_ser_w.astype(bf16), head_ser_b.reshape(1, EMO), wsr, bsr)

    return (lser, la[:, :, :VOCAB], lsr[:, :SPK])


# speedup vs baseline: 3.4632x; 1.4202x over previous
import jax
import jax.numpy as jnp
from jax.experimental import pallas as pl
from jax.experimental.pallas import tpu as pltpu

B, T, H = 8, 512, 1024
E, BN, RH = 8, 256, 256
VOCAB, EMO, SPK = 1000, 8, 1000
TOP_K = 2
TEMP = 2.0
VPAD = 1024


def _gelu(x):
    return 0.5 * x * (1.0 + jax.lax.erf(x * 0.7071067811865476))


def _router_kernel(h_ref, m_ref, te_ref, w1_ref, b1_ref, w2_ref, b2_ref,
                   pooled_ref, topi_ref, gates_ref):
    m = m_ref[...]
    den = jnp.maximum(m.sum(axis=1, keepdims=True), 1.0)
    num = (h_ref[...] * m[:, :, None]).sum(axis=1)
    pooled = num / den
    pooled_ref[...] = pooled
    for rid in range(3):
        x = pooled + te_ref[rid][None, :]
        rh = jnp.dot(x.astype(jnp.bfloat16), w1_ref[rid],
                     preferred_element_type=jnp.float32) + b1_ref[rid][None, :]
        rh = _gelu(rh)
        logits = (jnp.dot(rh.astype(jnp.bfloat16), w2_ref[rid],
                          preferred_element_type=jnp.float32)
                  + b2_ref[rid][None, :]) / TEMP
        i1 = jnp.argmax(logits, axis=-1).astype(jnp.int32)
        v1 = jnp.max(logits, axis=-1)
        col = jax.lax.broadcasted_iota(jnp.int32, (B, E), 1)
        masked = jnp.where(col == i1[:, None], -jnp.inf, logits)
        i2 = jnp.argmax(masked, axis=-1).astype(jnp.int32)
        v2 = jnp.max(masked, axis=-1)
        g1 = jax.nn.sigmoid(v1 - v2)
        topi_ref[rid, :, 0] = i1
        topi_ref[rid, :, 1] = i2
        gates_ref[rid, :, 0] = g1
        gates_ref[rid, :, 1] = 1.0 - g1


def _fused_kernel(ti_ref, gv_ref,
                  h_ref, m_ref, mfull_ref, te_ref, wd_ref, db_ref, wu_ref,
                  ub_ref, wasr_ref, basr_ref, pooled_ref, wser_ref, bser_ref,
                  wsr_ref, bsr_ref,
                  la_ref, lser_ref, lsr_ref, pmix_ref):
    b = pl.program_id(0)
    hb = h_ref[0]
    mcol = m_ref[0, 0][:, None]
    for rid in range(3):
        x = (hb + te_ref[rid][None, :]).astype(jnp.bfloat16)
        c = None
        for k in range(2):
            s = (rid * B + b) * 2 + k
            idx = ti_ref[s]
            g = gv_ref[s]
            d = jnp.dot(x, wd_ref[idx],
                        preferred_element_type=jnp.float32) + db_ref[idx]
            a = _gelu(d)
            u = jnp.dot((g * a).astype(jnp.bfloat16), wu_ref[idx],
                        preferred_element_type=jnp.float32) + g * ub_ref[idx]
            c = u if k == 0 else c + u
        if rid == 0:
            asr = hb + c
            la_ref[0] = (jnp.dot(asr.astype(jnp.bfloat16), wasr_ref[...],
                                 preferred_element_type=jnp.float32)
                         + basr_ref[0][None, :])
        else:
            pc = (c * mcol).sum(axis=0)
            pmix_ref[rid - 1, pl.ds(b, 1)] = pc[None, :]

    @pl.when(b == B - 1)
    def _():
        den = jnp.maximum(mfull_ref[...].sum(axis=1, keepdims=True), 1.0)
        ps = pooled_ref[...] + pmix_ref[0] / den
        pr = pooled_ref[...] + pmix_ref[1] / den
        lser_ref[...] = (jnp.dot(ps.astype(jnp.bfloat16), wser_ref[...],
                                 preferred_element_type=jnp.float32)
                         + bser_ref[0][None, :])
        lsr_ref[...] = (jnp.dot(pr.astype(jnp.bfloat16), wsr_ref[...],
                                preferred_element_type=jnp.float32)
                        + bsr_ref[0][None, :])


def kernel(input_values, attention_mask, task_embed, exp_down_w, exp_down_b,
           exp_up_w, exp_up_b, r_w1, r_b1, r_w2, r_b2, head_asr_w, head_asr_b,
           head_ser_w, head_ser_b, head_sr_w, head_sr_b):
    h = input_values
    mask = attention_mask
    f32 = jnp.float32
    bf16 = jnp.bfloat16

    pooled, topi, gates = pl.pallas_call(
        _router_kernel,
        grid=(1,),
        in_specs=[
            pl.BlockSpec((B, T, H), lambda i: (0, 0, 0)),
            pl.BlockSpec((B, T), lambda i: (0, 0)),
            pl.BlockSpec((3, H), lambda i: (0, 0)),
            pl.BlockSpec((3, H, RH), lambda i: (0, 0, 0)),
            pl.BlockSpec((3, RH), lambda i: (0, 0)),
            pl.BlockSpec((3, RH, E), lambda i: (0, 0, 0)),
            pl.BlockSpec((3, E), lambda i: (0, 0)),
        ],
        out_specs=[
            pl.BlockSpec((B, H), lambda i: (0, 0)),
            pl.BlockSpec((3, B, 2), lambda i: (0, 0, 0)),
            pl.BlockSpec((3, B, 2), lambda i: (0, 0, 0)),
        ],
        out_shape=[
            jax.ShapeDtypeStruct((B, H), f32),
            jax.ShapeDtypeStruct((3, B, 2), jnp.int32),
            jax.ShapeDtypeStruct((3, B, 2), f32),
        ],
    )(h, mask, task_embed, r_w1.astype(bf16), r_b1, r_w2.astype(bf16), r_b2)

    topi_flat = topi.reshape(-1)
    gates_flat = gates.reshape(-1)

    wasr = jnp.zeros((H, VPAD), bf16).at[:, :VOCAB].set(head_asr_w.astype(bf16))
    basr = jnp.zeros((1, VPAD), f32).at[0, :VOCAB].set(head_asr_b)
    wsr = jnp.zeros((H, VPAD), bf16).at[:, :SPK].set(head_sr_w.astype(bf16))
    bsr = jnp.zeros((1, VPAD), f32).at[0, :SPK].set(head_sr_b)

    const3 = pl.BlockSpec((3, H), lambda b, ti, gv: (0, 0))
    la, lser, lsr = pl.pallas_call(
        _fused_kernel,
        grid_spec=pltpu.PrefetchScalarGridSpec(
            num_scalar_prefetch=2,
            grid=(B,),
            in_specs=[
                pl.BlockSpec((1, T, H), lambda b, ti, gv: (b, 0, 0)),
                pl.BlockSpec((1, 1, T), lambda b, ti, gv: (b, 0, 0)),
                pl.BlockSpec((B, T), lambda b, ti, gv: (0, 0)),
                const3,
                pl.BlockSpec((E, H, BN), lambda b, ti, gv: (0, 0, 0)),
                pl.BlockSpec((E, 1, BN), lambda b, ti, gv: (0, 0, 0)),
                pl.BlockSpec((E, BN, H), lambda b, ti, gv: (0, 0, 0)),
                pl.BlockSpec((E, 1, H), lambda b, ti, gv: (0, 0, 0)),
                pl.BlockSpec((H, VPAD), lambda b, ti, gv: (0, 0)),
                pl.BlockSpec((1, VPAD), lambda b, ti, gv: (0, 0)),
                pl.BlockSpec((B, H), lambda b, ti, gv: (0, 0)),
                pl.BlockSpec((H, EMO), lambda b, ti, gv: (0, 0)),
                pl.BlockSpec((1, EMO), lambda b, ti, gv: (0, 0)),
                pl.BlockSpec((H, VPAD), lambda b, ti, gv: (0, 0)),
                pl.BlockSpec((1, VPAD), lambda b, ti, gv: (0, 0)),
            ],
            out_specs=[
                pl.BlockSpec((1, T, VPAD), lambda b, ti, gv: (b, 0, 0)),
                pl.BlockSpec((B, EMO), lambda b, ti, gv: (0, 0)),
                pl.BlockSpec((B, VPAD), lambda b, ti, gv: (0, 0)),
            ],
            scratch_shapes=[pltpu.VMEM((2, B, H), f32)],
        ),
        out_shape=[
            jax.ShapeDtypeStruct((B, T, VPAD), f32),
            jax.ShapeDtypeStruct((B, EMO), f32),
            jax.ShapeDtypeStruct((B, VPAD), f32),
        ],
    )(topi_flat, gates_flat, h, mask.reshape(B, 1, T), mask, task_embed,
      exp_down_w.astype(bf16), exp_down_b.reshape(E, 1, BN),
      exp_up_w.astype(bf16), exp_up_b.reshape(E, 1, H),
      wasr, basr, pooled, head_ser_w.astype(bf16), head_ser_b.reshape(1, EMO),
      wsr, bsr)

    return (lser, la[:, :, :VOCAB], lsr[:, :SPK])


# speedup vs baseline: 4.2561x; 1.2289x over previous
import jax
import jax.numpy as jnp
from jax.experimental import pallas as pl
from jax.experimental.pallas import tpu as pltpu

B, T, H = 8, 512, 1024
E, BN, RH = 8, 256, 256
VOCAB, EMO, SPK = 1000, 8, 1000
TOP_K = 2
TEMP = 2.0


def _gelu(x):
    return 0.5 * x * (1.0 + jax.lax.erf(x * 0.7071067811865476))


def _router_kernel(h_ref, m_ref, te_ref, w1_ref, b1_ref, w2_ref, b2_ref,
                   pooled_ref, topi_ref, gates_ref):
    m = m_ref[...]
    den = jnp.maximum(m.sum(axis=1, keepdims=True), 1.0)
    num = (h_ref[...] * m[:, :, None]).sum(axis=1)
    pooled = num / den
    pooled_ref[...] = pooled
    for rid in range(3):
        x = pooled + te_ref[rid][None, :]
        rh = jnp.dot(x.astype(jnp.bfloat16), w1_ref[rid].astype(jnp.bfloat16),
                     preferred_element_type=jnp.float32) + b1_ref[rid][None, :]
        rh = _gelu(rh)
        logits = (jnp.dot(rh.astype(jnp.bfloat16), w2_ref[rid].astype(jnp.bfloat16),
                          preferred_element_type=jnp.float32)
                  + b2_ref[rid][None, :]) / TEMP
        i1 = jnp.argmax(logits, axis=-1).astype(jnp.int32)
        v1 = jnp.max(logits, axis=-1)
        col = jax.lax.broadcasted_iota(jnp.int32, (B, E), 1)
        masked = jnp.where(col == i1[:, None], -jnp.inf, logits)
        i2 = jnp.argmax(masked, axis=-1).astype(jnp.int32)
        v2 = jnp.max(masked, axis=-1)
        g1 = jax.nn.sigmoid(v1 - v2)
        topi_ref[rid, :, 0] = i1
        topi_ref[rid, :, 1] = i2
        gates_ref[rid, :, 0] = g1
        gates_ref[rid, :, 1] = 1.0 - g1


def _fused_kernel(ti_ref, gv_ref,
                  h_ref, m_ref, mfull_ref, te_ref, wd_ref, db_ref, wu_ref,
                  ub_ref, wasr_ref, basr_ref, pooled_ref, wser_ref, bser_ref,
                  wsr_ref, bsr_ref,
                  la_ref, lser_ref, lsr_ref,
                  pmix_ref, wdbf_ref, wubf_ref, wabf_ref, wsbf_ref):
    b = pl.program_id(0)

    @pl.when(b == 0)
    def _():
        wdbf_ref[...] = wd_ref[...].astype(jnp.bfloat16)
        wubf_ref[...] = wu_ref[...].astype(jnp.bfloat16)
        wabf_ref[...] = wasr_ref[...].astype(jnp.bfloat16)
        wsbf_ref[...] = wsr_ref[...].astype(jnp.bfloat16)

    hb = h_ref[0]
    mcol = m_ref[0, 0][:, None]
    for rid in range(3):
        x = (hb + te_ref[rid][None, :]).astype(jnp.bfloat16)
        c = None
        for k in range(2):
            s = (rid * B + b) * 2 + k
            idx = ti_ref[s]
            g = gv_ref[s]
            d = jnp.dot(x, wdbf_ref[idx],
                        preferred_element_type=jnp.float32) + db_ref[idx]
            a = _gelu(d)
            u = jnp.dot((g * a).astype(jnp.bfloat16), wubf_ref[idx],
                        preferred_element_type=jnp.float32) + g * ub_ref[idx]
            c = u if k == 0 else c + u
        if rid == 0:
            asr = hb + c
            la_ref[0] = (jnp.dot(asr.astype(jnp.bfloat16), wabf_ref[...],
                                 preferred_element_type=jnp.float32)
                         + basr_ref[0][None, :])
        else:
            pc = (c * mcol).sum(axis=0)
            pmix_ref[rid - 1, pl.ds(b, 1)] = pc[None, :]

    @pl.when(b == B - 1)
    def _():
        den = jnp.maximum(mfull_ref[...].sum(axis=1, keepdims=True), 1.0)
        ps = pooled_ref[...] + pmix_ref[0] / den
        pr = pooled_ref[...] + pmix_ref[1] / den
        lser_ref[...] = (jnp.dot(ps.astype(jnp.bfloat16),
                                 wser_ref[...].astype(jnp.bfloat16),
                                 preferred_element_type=jnp.float32)
                         + bser_ref[0][None, :])
        lsr_ref[...] = (jnp.dot(pr.astype(jnp.bfloat16), wsbf_ref[...],
                                preferred_element_type=jnp.float32)
                        + bsr_ref[0][None, :])


def kernel(input_values, attention_mask, task_embed, exp_down_w, exp_down_b,
           exp_up_w, exp_up_b, r_w1, r_b1, r_w2, r_b2, head_asr_w, head_asr_b,
           head_ser_w, head_ser_b, head_sr_w, head_sr_b):
    h = input_values
    mask = attention_mask
    f32 = jnp.float32
    bf16 = jnp.bfloat16

    pooled, topi, gates = pl.pallas_call(
        _router_kernel,
        grid=(1,),
        in_specs=[
            pl.BlockSpec((B, T, H), lambda i: (0, 0, 0)),
            pl.BlockSpec((B, T), lambda i: (0, 0)),
            pl.BlockSpec((3, H), lambda i: (0, 0)),
            pl.BlockSpec((3, H, RH), lambda i: (0, 0, 0)),
            pl.BlockSpec((3, RH), lambda i: (0, 0)),
            pl.BlockSpec((3, RH, E), lambda i: (0, 0, 0)),
            pl.BlockSpec((3, E), lambda i: (0, 0)),
        ],
        out_specs=[
            pl.BlockSpec((B, H), lambda i: (0, 0)),
            pl.BlockSpec((3, B, 2), lambda i: (0, 0, 0)),
            pl.BlockSpec((3, B, 2), lambda i: (0, 0, 0)),
        ],
        out_shape=[
            jax.ShapeDtypeStruct((B, H), f32),
            jax.ShapeDtypeStruct((3, B, 2), jnp.int32),
            jax.ShapeDtypeStruct((3, B, 2), f32),
        ],
    )(h, mask, task_embed, r_w1, r_b1, r_w2, r_b2)

    topi_flat = topi.reshape(-1)
    gates_flat = gates.reshape(-1)

    la, lser, lsr = pl.pallas_call(
        _fused_kernel,
        grid_spec=pltpu.PrefetchScalarGridSpec(
            num_scalar_prefetch=2,
            grid=(B,),
            in_specs=[
                pl.BlockSpec((1, T, H), lambda b, ti, gv: (b, 0, 0)),
                pl.BlockSpec((1, 1, T), lambda b, ti, gv: (b, 0, 0)),
                pl.BlockSpec((B, T), lambda b, ti, gv: (0, 0)),
                pl.BlockSpec((3, H), lambda b, ti, gv: (0, 0)),
                pl.BlockSpec((E, H, BN), lambda b, ti, gv: (0, 0, 0)),
                pl.BlockSpec((E, 1, BN), lambda b, ti, gv: (0, 0, 0)),
                pl.BlockSpec((E, BN, H), lambda b, ti, gv: (0, 0, 0)),
                pl.BlockSpec((E, 1, H), lambda b, ti, gv: (0, 0, 0)),
                pl.BlockSpec((H, VOCAB), lambda b, ti, gv: (0, 0)),
                pl.BlockSpec((1, VOCAB), lambda b, ti, gv: (0, 0)),
                pl.BlockSpec((B, H), lambda b, ti, gv: (0, 0)),
                pl.BlockSpec((H, EMO), lambda b, ti, gv: (0, 0)),
                pl.BlockSpec((1, EMO), lambda b, ti, gv: (0, 0)),
                pl.BlockSpec((H, SPK), lambda b, ti, gv: (0, 0)),
                pl.BlockSpec((1, SPK), lambda b, ti, gv: (0, 0)),
            ],
            out_specs=[
                pl.BlockSpec((1, T, VOCAB), lambda b, ti, gv: (b, 0, 0)),
                pl.BlockSpec((B, EMO), lambda b, ti, gv: (0, 0)),
                pl.BlockSpec((B, SPK), lambda b, ti, gv: (0, 0)),
            ],
            scratch_shapes=[
                pltpu.VMEM((2, B, H), f32),
                pltpu.VMEM((E, H, BN), bf16),
                pltpu.VMEM((E, BN, H), bf16),
                pltpu.VMEM((H, VOCAB), bf16),
                pltpu.VMEM((H, SPK), bf16),
            ],
        ),
        out_shape=[
            jax.ShapeDtypeStruct((B, T, VOCAB), f32),
            jax.ShapeDtypeStruct((B, EMO), f32),
            jax.ShapeDtypeStruct((B, SPK), f32),
        ],
    )(topi_flat, gates_flat, h, mask.reshape(B, 1, T), mask, task_embed,
      exp_down_w, exp_down_b.reshape(E, 1, BN),
      exp_up_w, exp_up_b.reshape(E, 1, H),
      head_asr_w, head_asr_b.reshape(1, VOCAB), pooled,
      head_ser_w, head_ser_b.reshape(1, EMO),
      head_sr_w, head_sr_b.reshape(1, SPK))

    return (lser, la, lsr)


# speedup vs baseline: 4.4670x; 1.0496x over previous
import jax
import jax.numpy as jnp
from jax.experimental import pallas as pl
from jax.experimental.pallas import tpu as pltpu

B, T, H = 8, 512, 1024
E, BN, RH = 8, 256, 256
VOCAB, EMO, SPK = 1000, 8, 1000
TOP_K = 2
TEMP = 2.0


def _gelu(x):
    return 0.5 * x * (1.0 + jax.lax.erf(x * 0.7071067811865476))


def _router_kernel(h_ref, m_ref, te_ref, w1_ref, b1_ref, w2_ref, b2_ref,
                   pooled_ref, topi_ref, gates_ref):
    m = m_ref[...]
    den = jnp.maximum(m.sum(axis=1, keepdims=True), 1.0)
    num = (h_ref[...] * m[:, :, None]).sum(axis=1)
    pooled = num / den
    pooled_ref[...] = pooled
    for rid in range(3):
        x = pooled + te_ref[rid][None, :]
        rh = jnp.dot(x.astype(jnp.bfloat16), w1_ref[rid].astype(jnp.bfloat16),
                     preferred_element_type=jnp.float32) + b1_ref[rid][None, :]
        rh = _gelu(rh)
        logits = (jnp.dot(rh.astype(jnp.bfloat16), w2_ref[rid].astype(jnp.bfloat16),
                          preferred_element_type=jnp.float32)
                  + b2_ref[rid][None, :]) / TEMP
        i1 = jnp.argmax(logits, axis=-1).astype(jnp.int32)
        v1 = jnp.max(logits, axis=-1)
        col = jax.lax.broadcasted_iota(jnp.int32, (B, E), 1)
        masked = jnp.where(col == i1[:, None], -jnp.inf, logits)
        i2 = jnp.argmax(masked, axis=-1).astype(jnp.int32)
        v2 = jnp.max(masked, axis=-1)
        g1 = jax.nn.sigmoid(v1 - v2)
        topi_ref[rid, :, 0] = i1
        topi_ref[rid, :, 1] = i2
        gates_ref[rid, :, 0] = g1
        gates_ref[rid, :, 1] = 1.0 - g1


def _fused_kernel(ti_ref, gv_ref,
                  h_ref, m_ref, mfull_ref, te_ref, wd_ref, db_ref, wu_ref,
                  ub_ref, wasr_ref, basr_ref, pooled_ref, wser_ref, bser_ref,
                  wsr_ref, bsr_ref,
                  la_ref, lser_ref, lsr_ref,
                  pmix_ref, wdbf_ref, wubf_ref, wabf_ref, wsbf_ref):
    b = pl.program_id(0)

    @pl.when(b == 0)
    def _():
        wdbf_ref[...] = wd_ref[...].astype(jnp.bfloat16)
        wubf_ref[...] = wu_ref[...].astype(jnp.bfloat16)
        wabf_ref[...] = wasr_ref[...].astype(jnp.bfloat16)
        wsbf_ref[...] = wsr_ref[...].astype(jnp.bfloat16)

    hb = h_ref[0]
    mcol = m_ref[0, 0][:, None]
    for rid in range(3):
        x = (hb + te_ref[rid][None, :]).astype(jnp.bfloat16)
        c = None
        for k in range(2):
            idx = ti_ref[rid, b, k]
            g = gv_ref[rid, b, k]
            d = jnp.dot(x, wdbf_ref[idx],
                        preferred_element_type=jnp.float32) + db_ref[pl.ds(idx, 1)]
            a = _gelu(d)
            u = jnp.dot((g * a).astype(jnp.bfloat16), wubf_ref[idx],
                        preferred_element_type=jnp.float32) + g * ub_ref[pl.ds(idx, 1)]
            c = u if k == 0 else c + u
        if rid == 0:
            asr = hb + c
            la_ref[0] = (jnp.dot(asr.astype(jnp.bfloat16), wabf_ref[...],
                                 preferred_element_type=jnp.float32)
                         + basr_ref[...][None, :])
        else:
            pc = (c * mcol).sum(axis=0)
            pmix_ref[rid - 1, pl.ds(b, 1)] = pc[None, :]

    @pl.when(b == B - 1)
    def _():
        den = jnp.maximum(mfull_ref[...].sum(axis=1, keepdims=True), 1.0)
        ps = pooled_ref[...] + pmix_ref[0] / den
        pr = pooled_ref[...] + pmix_ref[1] / den
        lser_ref[...] = (jnp.dot(ps.astype(jnp.bfloat16),
                                 wser_ref[...].astype(jnp.bfloat16),
                                 preferred_element_type=jnp.float32)
                         + bser_ref[...][None, :])
        lsr_ref[...] = (jnp.dot(pr.astype(jnp.bfloat16), wsbf_ref[...],
                                preferred_element_type=jnp.float32)
                        + bsr_ref[...][None, :])


def kernel(input_values, attention_mask, task_embed, exp_down_w, exp_down_b,
           exp_up_w, exp_up_b, r_w1, r_b1, r_w2, r_b2, head_asr_w, head_asr_b,
           head_ser_w, head_ser_b, head_sr_w, head_sr_b):
    h = input_values
    mask = attention_mask
    f32 = jnp.float32
    bf16 = jnp.bfloat16

    pooled, topi, gates = pl.pallas_call(
        _router_kernel,
        grid=(1,),
        in_specs=[
            pl.BlockSpec((B, T, H), lambda i: (0, 0, 0)),
            pl.BlockSpec((B, T), lambda i: (0, 0)),
            pl.BlockSpec((3, H), lambda i: (0, 0)),
            pl.BlockSpec((3, H, RH), lambda i: (0, 0, 0)),
            pl.BlockSpec((3, RH), lambda i: (0, 0)),
            pl.BlockSpec((3, RH, E), lambda i: (0, 0, 0)),
            pl.BlockSpec((3, E), lambda i: (0, 0)),
        ],
        out_specs=[
            pl.BlockSpec((B, H), lambda i: (0, 0)),
            pl.BlockSpec((3, B, 2), lambda i: (0, 0, 0)),
            pl.BlockSpec((3, B, 2), lambda i: (0, 0, 0)),
        ],
        out_shape=[
            jax.ShapeDtypeStruct((B, H), f32),
            jax.ShapeDtypeStruct((3, B, 2), jnp.int32),
            jax.ShapeDtypeStruct((3, B, 2), f32),
        ],
    )(h, mask, task_embed, r_w1, r_b1, r_w2, r_b2)

    la, lser, lsr = pl.pallas_call(
        _fused_kernel,
        grid_spec=pltpu.PrefetchScalarGridSpec(
            num_scalar_prefetch=2,
            grid=(B,),
            in_specs=[
                pl.BlockSpec((1, T, H), lambda b, ti, gv: (b, 0, 0)),
                pl.BlockSpec((1, 1, T), lambda b, ti, gv: (b, 0, 0)),
                pl.BlockSpec((B, T), lambda b, ti, gv: (0, 0)),
                pl.BlockSpec((3, H), lambda b, ti, gv: (0, 0)),
                pl.BlockSpec((E, H, BN), lambda b, ti, gv: (0, 0, 0)),
                pl.BlockSpec((E, BN), lambda b, ti, gv: (0, 0)),
                pl.BlockSpec((E, BN, H), lambda b, ti, gv: (0, 0, 0)),
                pl.BlockSpec((E, H), lambda b, ti, gv: (0, 0)),
                pl.BlockSpec((H, VOCAB), lambda b, ti, gv: (0, 0)),
                pl.BlockSpec((VOCAB,), lambda b, ti, gv: (0,)),
                pl.BlockSpec((B, H), lambda b, ti, gv: (0, 0)),
                pl.BlockSpec((H, EMO), lambda b, ti, gv: (0, 0)),
                pl.BlockSpec((EMO,), lambda b, ti, gv: (0,)),
                pl.BlockSpec((H, SPK), lambda b, ti, gv: (0, 0)),
                pl.BlockSpec((SPK,), lambda b, ti, gv: (0,)),
            ],
            out_specs=[
                pl.BlockSpec((1, T, VOCAB), lambda b, ti, gv: (b, 0, 0)),
                pl.BlockSpec((B, EMO), lambda b, ti, gv: (0, 0)),
                pl.BlockSpec((B, SPK), lambda b, ti, gv: (0, 0)),
            ],
            scratch_shapes=[
                pltpu.VMEM((2, B, H), f32),
                pltpu.VMEM((E, H, BN), bf16),
                pltpu.VMEM((E, BN, H), bf16),
                pltpu.VMEM((H, VOCAB), bf16),
                pltpu.VMEM((H, SPK), bf16),
            ],
        ),
        out_shape=[
            jax.ShapeDtypeStruct((B, T, VOCAB), f32),
            jax.ShapeDtypeStruct((B, EMO), f32),
            jax.ShapeDtypeStruct((B, SPK), f32),
        ],
    )(topi, gates, h, mask.reshape(B, 1, T), mask, task_embed,
      exp_down_w, exp_down_b, exp_up_w, exp_up_b,
      head_asr_w, head_asr_b, pooled,
      head_ser_w, head_ser_b, head_sr_w, head_sr_b)

    return (lser, la, lsr)
